# baseline (speedup 1.0000x reference)
import jax
import jax.numpy as jnp
from jax import lax
from jax.experimental import pallas as pl
from jax.experimental.pallas import tpu as pltpu
from jax.experimental.pallas import tpu_sc as plsc

N = 10000
E = 320000
D_IN = 128
H = 16
C = 2

NC = 2
NS = 16
NW = NC * NS
NPAD = 10112
CH = 40
NCH = 252
EPT = NCH * CH
EPADT = EPT * NW
RPT = NPAD // NS
NBUF = 3

_SC_PARAMS = pltpu.CompilerParams(use_tc_tiling_on_sc=False)
_SC_MESH = plsc.VectorSubcoreMesh(core_axis_name="c", subcore_axis_name="s")


def _make_adj(width):
    def body(t_hbm, src_hbm, dst_hbm, z_hbm, out_hbm,
             srcb, dstb, rows, acc, sg):
        cid = lax.axis_index("c")
        sid = lax.axis_index("s")
        wid = cid * NS + sid

        pltpu.sync_copy(z_hbm.at[pl.ds(sid * RPT, RPT)],
                        acc.at[pl.ds(sid * RPT, RPT)])

        pltpu.sync_copy(src_hbm.at[wid], srcb)
        pltpu.sync_copy(dst_hbm.at[wid], dstb)

        plsc.subcore_barrier()

        for b in range(NBUF):
            pltpu.async_copy(t_hbm.at[srcb.at[b]], rows[b], sg[b])

        def _step(i, carry):
            c0 = NBUF * i
            for b in range(NBUF):
                pltpu.make_async_copy(t_hbm.at[srcb.at[c0 + b]],
                                      rows[b], sg[b]).wait()
                pltpu.sync_copy(rows[b], acc.at[dstb.at[c0 + b]], add=True)
                pltpu.async_copy(t_hbm.at[srcb.at[c0 + NBUF + b]],
                                 rows[b], sg[b])
            return carry
        lax.fori_loop(0, NCH // NBUF, _step, 0)

        for b in range(NBUF):
            pltpu.make_async_copy(t_hbm.at[srcb.at[NCH + b]],
                                  rows[b], sg[b]).wait()

        plsc.subcore_barrier()
        pltpu.sync_copy(acc.at[pl.ds(sid * RPT, RPT)],
                        out_hbm.at[pl.ds(cid * NPAD + sid * RPT, RPT)])

    return pl.kernel(
        body,
        out_type=jax.ShapeDtypeStruct((2 * NPAD, width), jnp.float32),
        mesh=_SC_MESH,
        scratch_types=[
            pltpu.VMEM((NCH + NBUF, CH), jnp.int32),
            pltpu.VMEM((NCH + NBUF, CH), jnp.int32),
            [pltpu.VMEM((CH, width), jnp.float32)] * NBUF,
            pltpu.VMEM_SHARED((NPAD, width), jnp.float32),
            [pltpu.SemaphoreType.DMA] * NBUF,
        ],
        compiler_params=_SC_PARAMS,
    )


def _make_deg(width):
    def body(dst_hbm, ones_hbm, z_hbm, out_hbm, dstb, onesb, acc, ss):
        cid = lax.axis_index("c")
        sid = lax.axis_index("s")
        wid = cid * NS + sid

        pltpu.sync_copy(z_hbm.at[pl.ds(sid * RPT, RPT)],
                        acc.at[pl.ds(sid * RPT, RPT)])
        pltpu.sync_copy(ones_hbm, onesb)
        pltpu.sync_copy(dst_hbm.at[wid], dstb)

        plsc.subcore_barrier()

        for b in range(NBUF):
            pltpu.make_async_copy(onesb, acc.at[dstb.at[b]],
                                  ss[b]).start(add=True)

        def _step(i, carry):
            c0 = NBUF * i
            for b in range(NBUF):
                pltpu.make_async_copy(onesb, acc.at[dstb.at[c0 + b]],
                                      ss[b]).wait()
                pltpu.make_async_copy(onesb, acc.at[dstb.at[c0 + NBUF + b]],
                                      ss[b]).start(add=True)
            return carry
        lax.fori_loop(0, NCH // NBUF - 1, _step, 0)

        c0 = NCH - NBUF
        for b in range(NBUF):
            pltpu.make_async_copy(onesb, acc.at[dstb.at[c0 + b]],
                                  ss[b]).wait()

        plsc.subcore_barrier()
        pltpu.sync_copy(acc.at[pl.ds(sid * RPT, RPT)],
                        out_hbm.at[pl.ds(cid * NPAD + sid * RPT, RPT)])

    return pl.kernel(
        body,
        out_type=jax.ShapeDtypeStruct((2 * NPAD, width), jnp.float32),
        mesh=_SC_MESH,
        scratch_types=[
            pltpu.VMEM((NCH + NBUF, CH), jnp.int32),
            pltpu.VMEM((CH, width), jnp.float32),
            pltpu.VMEM_SHARED((NPAD, width), jnp.float32),
            [pltpu.SemaphoreType.DMA] * NBUF,
        ],
        compiler_params=_SC_PARAMS,
    )


_adj16 = _make_adj(16)
_adj8 = _make_adj(8)
_deg8 = _make_deg(8)


RB = 1264


def _prep1_body(dp0, dp1, x, w, b, t3, a1, a2, u0, dinv, d2, dinvh, d2h):
    deg = dp0[...] + dp1[...]
    di8 = jnp.where(deg > 0, lax.rsqrt(jnp.maximum(deg, 1e-12)), 0.0)
    di = jnp.concatenate([di8, di8], axis=1)
    u = jnp.dot(x[...], w[...], preferred_element_type=jnp.float32)
    t3[...] = di * u[:, 48:64]
    a1[...] = di * u[:, 32:48]
    a2[...] = di * u[:, 16:32]
    u0[...] = u[:, 0:16] + b[...]
    dinv[...] = di
    d2[...] = di * di
    dinvh[...] = di8
    d2h[...] = di8 * di8


def _mid_body(sp0, sp1, a, d2, t):
    t[...] = a[...] + d2[...] * (sp0[...] + sp1[...])


def _l2prep_body(sp0, sp1, u0, dinv, dinvh, w, wp, t3, a1, a2, v0):
    h = u0[...] + dinv[...] * (sp0[...] + sp1[...])
    h = jnp.maximum(h, 0.0)
    vp = jnp.dot(h, wp[...], preferred_element_type=jnp.float32)
    t3[...] = dinvh[...] * vp[:, 16:24]
    a1[...] = dinvh[...] * vp[:, 8:16]
    a2[...] = dinvh[...] * vp[:, 0:8]
    v0[...] = jnp.dot(h, w[...], preferred_element_type=jnp.float32)


def _final_body(sp0, sp1, v0, dinvh, b, out):
    o = v0[...] + dinvh[:, 0:2] * (sp0[:, 0:2] + sp1[:, 0:2]) + b[...]
    o0 = o[:, 0:1]
    o1 = o[:, 1:2]
    m = jnp.maximum(o0, o1)
    lse = m + jnp.log(jnp.exp(o0 - m) + jnp.exp(o1 - m))
    out[...] = o - lse


def _rows(width):
    return pl.BlockSpec((RB, width), lambda i: (i, 0))


def _full(r, c):
    return pl.BlockSpec((r, c), lambda i: (0, 0))


def _tc_call(body, in_specs, out_widths):
    return pl.pallas_call(
        body,
        grid=(NPAD // RB,),
        in_specs=in_specs,
        out_specs=[_rows(w) for w in out_widths],
        out_shape=[jax.ShapeDtypeStruct((NPAD, w), jnp.float32)
                   for w in out_widths],
    )


@jax.jit
def kernel(x, edge_index, W1, b1, W2, b2):
    f32 = jnp.float32

    x_p = jnp.zeros((NPAD, D_IN), f32).at[:N].set(x)
    pad = jnp.full((EPADT - E,), NPAD - 1, jnp.int32)
    zc = jnp.zeros((NW, NBUF, CH), jnp.int32)
    src3 = jnp.concatenate(
        [jnp.concatenate([edge_index[0], pad]).reshape(NW, NCH, CH), zc], 1)
    dst3 = jnp.concatenate(
        [jnp.concatenate([edge_index[1], pad]).reshape(NW, NCH, CH), zc], 1)

    w1r = jnp.concatenate([W1[k] for k in range(4)], axis=1)
    w2p = jnp.zeros((H, 24), f32)
    w2p = w2p.at[:, 0:2].set(W2[1]).at[:, 8:10].set(W2[2]) \
             .at[:, 16:18].set(W2[3])
    b1r = jnp.broadcast_to(b1[None, :], (NPAD, H)).astype(f32)
    b2r = jnp.broadcast_to(b2[None, :], (NPAD, C)).astype(f32)
    z16 = jnp.zeros((NPAD, 16), f32)
    z8 = jnp.zeros((NPAD, 8), f32)
    ones8 = jnp.ones((CH, 8), f32)

    dp = _deg8(dst3, ones8, z8)
    prep1 = _tc_call(
        _prep1_body,
        [_rows(8), _rows(8), _rows(D_IN), _full(D_IN, 64), _rows(16)],
        [16, 16, 16, 16, 16, 16, 8, 8])
    t3, a1, a2, u0, dinv, d2, dinvh, d2h = prep1(
        dp[:NPAD], dp[NPAD:], x_p, w1r, b1r)

    mid16 = _tc_call(_mid_body, [_rows(16)] * 4, [16])
    s = _adj16(t3, src3, dst3, z16)
    t = mid16(s[:NPAD], s[NPAD:], a1, d2)[0]
    s = _adj16(t, src3, dst3, z16)
    t = mid16(s[:NPAD], s[NPAD:], a2, d2)[0]
    s = _adj16(t, src3, dst3, z16)

    l2prep = _tc_call(
        _l2prep_body,
        [_rows(16)] * 4 + [_rows(8), _full(H, C), _full(H, 24)],
        [8, 8, 8, C])
    t3b, a1b, a2b, v0 = l2prep(s[:NPAD], s[NPAD:], u0, dinv, dinvh,
                               W2[0].astype(f32), w2p)

    mid8 = _tc_call(_mid_body, [_rows(8)] * 4, [8])
    s = _adj8(t3b, src3, dst3, z8)
    t = mid8(s[:NPAD], s[NPAD:], a1b, d2h)[0]
    s = _adj8(t, src3, dst3, z8)
    t = mid8(s[:NPAD], s[NPAD:], a2b, d2h)[0]
    s = _adj8(t, src3, dst3, z8)

    fin = _tc_call(_final_body,
                   [_rows(8), _rows(8), _rows(C), _rows(8), _rows(C)],
                   [C])
    out = fin(s[:NPAD], s[NPAD:], v0, dinvh, b2r)[0]
    return out[:N]


# speedup vs baseline: 1.0664x; 1.0664x over previous
import jax
import jax.numpy as jnp
from jax import lax
from jax.experimental import pallas as pl
from jax.experimental.pallas import tpu as pltpu
from jax.experimental.pallas import tpu_sc as plsc

N = 10000
E = 320000
D_IN = 128
H = 16
C = 2

NC = 2
NS = 16
NW = NC * NS
NPAD = 10112
CH = 64
NCH = 159
EPT = NCH * CH
EPADT = EPT * NW
RPT = NPAD // NS
NBUF = 3

_SC_PARAMS = pltpu.CompilerParams(use_tc_tiling_on_sc=False)
_SC_MESH = plsc.VectorSubcoreMesh(core_axis_name="c", subcore_axis_name="s")


def _make_adj(width):
    def body(t_hbm, src_hbm, dst_hbm, z_hbm, out_hbm,
             srcb, dstb, rows, acc, sg):
        cid = lax.axis_index("c")
        sid = lax.axis_index("s")
        wid = cid * NS + sid

        pltpu.sync_copy(z_hbm.at[pl.ds(sid * RPT, RPT)],
                        acc.at[pl.ds(sid * RPT, RPT)])

        pltpu.sync_copy(src_hbm.at[wid], srcb)
        pltpu.sync_copy(dst_hbm.at[wid], dstb)

        plsc.subcore_barrier()

        for b in range(NBUF):
            pltpu.async_copy(t_hbm.at[srcb.at[b]], rows[b], sg[b])

        def _step(i, carry):
            c0 = NBUF * i
            for b in range(NBUF):
                pltpu.make_async_copy(t_hbm.at[srcb.at[c0 + b]],
                                      rows[b], sg[b]).wait()
                pltpu.sync_copy(rows[b], acc.at[dstb.at[c0 + b]], add=True)
                pltpu.async_copy(t_hbm.at[srcb.at[c0 + NBUF + b]],
                                 rows[b], sg[b])
            return carry
        lax.fori_loop(0, NCH // NBUF, _step, 0)

        for b in range(NBUF):
            pltpu.make_async_copy(t_hbm.at[srcb.at[NCH + b]],
                                  rows[b], sg[b]).wait()

        plsc.subcore_barrier()
        pltpu.sync_copy(acc.at[pl.ds(sid * RPT, RPT)],
                        out_hbm.at[pl.ds(cid * NPAD + sid * RPT, RPT)])

    return pl.kernel(
        body,
        out_type=jax.ShapeDtypeStruct((2 * NPAD, width), jnp.float32),
        mesh=_SC_MESH,
        scratch_types=[
            pltpu.VMEM((NCH + NBUF, CH), jnp.int32),
            pltpu.VMEM((NCH + NBUF, CH), jnp.int32),
            [pltpu.VMEM((CH, width), jnp.float32)] * NBUF,
            pltpu.VMEM_SHARED((NPAD, width), jnp.float32),
            [pltpu.SemaphoreType.DMA] * NBUF,
        ],
        compiler_params=_SC_PARAMS,
    )


def _make_deg(width):
    def body(dst_hbm, ones_hbm, z_hbm, out_hbm, dstb, onesb, acc, ss):
        cid = lax.axis_index("c")
        sid = lax.axis_index("s")
        wid = cid * NS + sid

        pltpu.sync_copy(z_hbm.at[pl.ds(sid * RPT, RPT)],
                        acc.at[pl.ds(sid * RPT, RPT)])
        pltpu.sync_copy(ones_hbm, onesb)
        pltpu.sync_copy(dst_hbm.at[wid], dstb)

        plsc.subcore_barrier()

        for b in range(NBUF):
            pltpu.make_async_copy(onesb, acc.at[dstb.at[b]],
                                  ss[b]).start(add=True)

        def _step(i, carry):
            c0 = NBUF * i
            for b in range(NBUF):
                pltpu.make_async_copy(onesb, acc.at[dstb.at[c0 + b]],
                                      ss[b]).wait()
                pltpu.make_async_copy(onesb, acc.at[dstb.at[c0 + NBUF + b]],
                                      ss[b]).start(add=True)
            return carry
        lax.fori_loop(0, NCH // NBUF - 1, _step, 0)

        c0 = NCH - NBUF
        for b in range(NBUF):
            pltpu.make_async_copy(onesb, acc.at[dstb.at[c0 + b]],
                                  ss[b]).wait()

        plsc.subcore_barrier()
        pltpu.sync_copy(acc.at[pl.ds(sid * RPT, RPT)],
                        out_hbm.at[pl.ds(cid * NPAD + sid * RPT, RPT)])

    return pl.kernel(
        body,
        out_type=jax.ShapeDtypeStruct((2 * NPAD, width), jnp.float32),
        mesh=_SC_MESH,
        scratch_types=[
            pltpu.VMEM((NCH + NBUF, CH), jnp.int32),
            pltpu.VMEM((CH, width), jnp.float32),
            pltpu.VMEM_SHARED((NPAD, width), jnp.float32),
            [pltpu.SemaphoreType.DMA] * NBUF,
        ],
        compiler_params=_SC_PARAMS,
    )


_adj16 = _make_adj(16)
_adj8 = _make_adj(8)
_deg8 = _make_deg(8)


RB = 1264


def _prep1_body(dp0, dp1, x, w, b, t3, a1, a2, u0, dinv, d2, dinvh, d2h):
    deg = dp0[...] + dp1[...]
    di8 = jnp.where(deg > 0, lax.rsqrt(jnp.maximum(deg, 1e-12)), 0.0)
    di = jnp.concatenate([di8, di8], axis=1)
    u = jnp.dot(x[...], w[...], preferred_element_type=jnp.float32)
    t3[...] = di * u[:, 48:64]
    a1[...] = di * u[:, 32:48]
    a2[...] = di * u[:, 16:32]
    u0[...] = u[:, 0:16] + b[...]
    dinv[...] = di
    d2[...] = di * di
    dinvh[...] = di8
    d2h[...] = di8 * di8


def _mid_body(sp0, sp1, a, d2, t):
    t[...] = a[...] + d2[...] * (sp0[...] + sp1[...])


def _l2prep_body(sp0, sp1, u0, dinv, dinvh, w, wp, t3, a1, a2, v0):
    h = u0[...] + dinv[...] * (sp0[...] + sp1[...])
    h = jnp.maximum(h, 0.0)
    vp = jnp.dot(h, wp[...], preferred_element_type=jnp.float32)
    t3[...] = dinvh[...] * vp[:, 16:24]
    a1[...] = dinvh[...] * vp[:, 8:16]
    a2[...] = dinvh[...] * vp[:, 0:8]
    v0[...] = jnp.dot(h, w[...], preferred_element_type=jnp.float32)


def _final_body(sp0, sp1, v0, dinvh, b, out):
    o = v0[...] + dinvh[:, 0:2] * (sp0[:, 0:2] + sp1[:, 0:2]) + b[...]
    o0 = o[:, 0:1]
    o1 = o[:, 1:2]
    m = jnp.maximum(o0, o1)
    lse = m + jnp.log(jnp.exp(o0 - m) + jnp.exp(o1 - m))
    out[...] = o - lse


def _rows(width):
    return pl.BlockSpec((RB, width), lambda i: (i, 0))


def _full(r, c):
    return pl.BlockSpec((r, c), lambda i: (0, 0))


def _tc_call(body, in_specs, out_widths):
    return pl.pallas_call(
        body,
        grid=(NPAD // RB,),
        in_specs=in_specs,
        out_specs=[_rows(w) for w in out_widths],
        out_shape=[jax.ShapeDtypeStruct((NPAD, w), jnp.float32)
                   for w in out_widths],
    )


@jax.jit
def kernel(x, edge_index, W1, b1, W2, b2):
    f32 = jnp.float32

    x_p = jnp.zeros((NPAD, D_IN), f32).at[:N].set(x)
    pad = jnp.full((EPADT - E,), NPAD - 1, jnp.int32)
    zc = jnp.zeros((NW, NBUF, CH), jnp.int32)
    src3 = jnp.concatenate(
        [jnp.concatenate([edge_index[0], pad]).reshape(NW, NCH, CH), zc], 1)
    dst3 = jnp.concatenate(
        [jnp.concatenate([edge_index[1], pad]).reshape(NW, NCH, CH), zc], 1)

    w1r = jnp.concatenate([W1[k] for k in range(4)], axis=1)
    w2p = jnp.zeros((H, 24), f32)
    w2p = w2p.at[:, 0:2].set(W2[1]).at[:, 8:10].set(W2[2]) \
             .at[:, 16:18].set(W2[3])
    b1r = jnp.broadcast_to(b1[None, :], (NPAD, H)).astype(f32)
    b2r = jnp.broadcast_to(b2[None, :], (NPAD, C)).astype(f32)
    z16 = jnp.zeros((NPAD, 16), f32)
    z8 = jnp.zeros((NPAD, 8), f32)
    ones8 = jnp.ones((CH, 8), f32)

    dp = _deg8(dst3, ones8, z8)
    prep1 = _tc_call(
        _prep1_body,
        [_rows(8), _rows(8), _rows(D_IN), _full(D_IN, 64), _rows(16)],
        [16, 16, 16, 16, 16, 16, 8, 8])
    t3, a1, a2, u0, dinv, d2, dinvh, d2h = prep1(
        dp[:NPAD], dp[NPAD:], x_p, w1r, b1r)

    mid16 = _tc_call(_mid_body, [_rows(16)] * 4, [16])
    s = _adj16(t3, src3, dst3, z16)
    t = mid16(s[:NPAD], s[NPAD:], a1, d2)[0]
    s = _adj16(t, src3, dst3, z16)
    t = mid16(s[:NPAD], s[NPAD:], a2, d2)[0]
    s = _adj16(t, src3, dst3, z16)

    l2prep = _tc_call(
        _l2prep_body,
        [_rows(16)] * 4 + [_rows(8), _full(H, C), _full(H, 24)],
        [8, 8, 8, C])
    t3b, a1b, a2b, v0 = l2prep(s[:NPAD], s[NPAD:], u0, dinv, dinvh,
                               W2[0].astype(f32), w2p)

    mid8 = _tc_call(_mid_body, [_rows(8)] * 4, [8])
    s = _adj8(t3b, src3, dst3, z8)
    t = mid8(s[:NPAD], s[NPAD:], a1b, d2h)[0]
    s = _adj8(t, src3, dst3, z8)
    t = mid8(s[:NPAD], s[NPAD:], a2b, d2h)[0]
    s = _adj8(t, src3, dst3, z8)

    fin = _tc_call(_final_body,
                   [_rows(8), _rows(8), _rows(C), _rows(8), _rows(C)],
                   [C])
    out = fin(s[:NPAD], s[NPAD:], v0, dinvh, b2r)[0]
    return out[:N]
